# baseline (speedup 1.0000x reference)
import functools

import jax
import jax.numpy as jnp
from jax import lax
from jax.experimental import pallas as pl
from jax.experimental.pallas import tpu as pltpu
from jax.experimental.pallas import tpu_sc as plsc

B, S, D = 16, 2048, 1024
L = 16


def _mid_body(table_hbm, lens_hbm, out_hbm, lens_v, idx_v, rows_v, sem):
    cid = lax.axis_index("c")
    sid = lax.axis_index("s")
    wid = sid * 2 + cid

    @pl.when(wid == 0)
    def _():
        pltpu.sync_copy(lens_hbm, lens_v)
        lens = lens_v[...]
        lane = lax.iota(jnp.int32, L)
        idx_v[...] = lax.shift_right_logical(lens, 1) + lane * S
        pltpu.async_copy(table_hbm.at[idx_v], rows_v, sem).wait()
        zeros = jnp.zeros((L,), jnp.float32)
        for b in range(B):
            len_b = lens[b]

            @pl.when(len_b == 0)
            def _zero(b=b):
                for j in range(D // L):
                    rows_v[b, j * L:(j + 1) * L] = zeros

        pltpu.sync_copy(rows_v, out_hbm)


def kernel(payload, seq_lens):
    table = payload.reshape(B * S, D)
    lens = seq_lens.astype(jnp.int32)
    mesh = plsc.VectorSubcoreMesh(core_axis_name="c", subcore_axis_name="s")
    k = functools.partial(
        pl.kernel,
        mesh=mesh,
        out_type=jax.ShapeDtypeStruct((B, D), jnp.float32),
        scratch_types=[
            pltpu.VMEM((L,), jnp.int32),
            pltpu.VMEM((L,), jnp.int32),
            pltpu.VMEM((B, D), jnp.float32),
            pltpu.SemaphoreType.DMA,
        ],
    )(_mid_body)
    return k(table, lens)


# speedup vs baseline: 1.1758x; 1.1758x over previous
import functools

import jax
import jax.numpy as jnp
from jax import lax
from jax.experimental import pallas as pl
from jax.experimental.pallas import tpu as pltpu
from jax.experimental.pallas import tpu_sc as plsc

B, S, D = 16, 2048, 1024
L = 16


def _mid_body(table_hbm, lens_hbm, out_hbm, lens_v, idx_v, rows_v, sem):
    cid = lax.axis_index("c")
    sid = lax.axis_index("s")
    wid = sid + cid

    @pl.when(wid == 0)
    def _():
        pltpu.sync_copy(lens_hbm, lens_v)
        lens = lens_v[...]
        lane = lax.iota(jnp.int32, L)
        idx_v[...] = lax.shift_right_logical(lens, 1) + lane * S
        pltpu.async_copy(table_hbm.at[idx_v], rows_v, sem).wait()
        zeros = jnp.zeros((L,), jnp.float32)
        for b in range(B):
            @pl.when(lens[b] == 0)
            def _zero(b=b):
                def chunk(j, c):
                    rows_v[b, pl.ds(j * L, L)] = zeros
                    return c

                lax.fori_loop(0, D // L, chunk, 0)

        pltpu.sync_copy(rows_v, out_hbm)


def kernel(payload, seq_lens):
    table = payload.reshape(B * S, D)
    lens = seq_lens.astype(jnp.int32)
    mesh = plsc.VectorSubcoreMesh(
        core_axis_name="c", subcore_axis_name="s", num_cores=1
    )
    k = functools.partial(
        pl.kernel,
        mesh=mesh,
        out_type=jax.ShapeDtypeStruct((B, D), jnp.float32),
        scratch_types=[
            pltpu.VMEM((L,), jnp.int32),
            pltpu.VMEM((L,), jnp.int32),
            pltpu.VMEM((B, D), jnp.float32),
            pltpu.SemaphoreType.DMA,
        ],
    )(_mid_body)
    return k(table, lens)
